# Initial kernel scaffold; baseline (speedup 1.0000x reference)
import functools

import jax
import jax.numpy as jnp
from jax import lax
from jax.experimental import pallas as pl
from jax.experimental.pallas import tpu as pltpu
from jax.experimental.pallas import tpu_sc as plsc

N = 10000
E = 320000
IN_CH_ = 128
HID_ = 50
OUT_ = 40
D = 64
L = 16
NC, NS = 2, 16
NW = NC * NS
EPT = E // NW
B = 80
NCHUNK = EPT // B
RB = 80
NRCHUNK = N // RB


def _dense_body(d0, x_ref, w_ref, attc_ref, h_ref, a_ref, m_ref):
    h = jnp.dot(x_ref[...], w_ref[...], preferred_element_type=jnp.float32,
                precision=lax.Precision.HIGHEST)
    col = lax.broadcasted_iota(jnp.int32, h.shape, 1)
    h = h + jnp.where(col == d0, 1.0, 0.0).astype(jnp.float32)
    h_ref[...] = h
    a = jnp.dot(h, attc_ref[...], preferred_element_type=jnp.float32,
                precision=lax.Precision.HIGHEST)
    a_ref[...] = a
    col2 = lax.broadcasted_iota(jnp.int32, a.shape, 1)
    a_masked = jnp.where(col2 == 0, a, -jnp.inf)
    m_ref[...] = jnp.max(a_masked)[None, None]


def _dense(x, Wp, attc, d0):
    return pl.pallas_call(
        functools.partial(_dense_body, d0),
        out_shape=[
            jax.ShapeDtypeStruct((N, D), jnp.float32),
            jax.ShapeDtypeStruct((N, 2), jnp.float32),
            jax.ShapeDtypeStruct((1, 1), jnp.float32),
        ],
    )(x, Wp, attc)


def _finalize_body(d0, n0_ref, n1_ref, h_ref, a_ref, m_ref, b_ref, g_ref):
    a = a_ref[...]
    col2 = lax.broadcasted_iota(jnp.int32, a.shape, 1)
    a_s = jnp.sum(jnp.where(col2 == 0, a, 0.0), axis=1, keepdims=True)
    a_d = jnp.sum(jnp.where(col2 == 1, a, 0.0), axis=1, keepdims=True)
    m = m_ref[0, 0]
    pre = a_s + a_d
    alpha = jnp.where(pre > 0, pre, 0.2 * pre)
    bnd = m + a_d
    shift = jnp.where(bnd > 0, bnd, 0.2 * bnd)
    p = jnp.exp(alpha - shift)
    num = n0_ref[...] + n1_ref[...] + p * h_ref[...]
    colD = lax.broadcasted_iota(jnp.int32, num.shape, 1)
    den = jnp.sum(jnp.where(colD == d0, num, 0.0), axis=1, keepdims=True)
    g = num / (den + 1e-16) + b_ref[...]
    g_ref[...] = jnp.maximum(g, 0.0)


def _finalize(n0, n1, h, a, m, bp, d0):
    return pl.pallas_call(
        functools.partial(_finalize_body, d0),
        out_shape=jax.ShapeDtypeStruct((N, D), jnp.float32),
    )(n0, n1, h, a, m, bp)


def _sc_body(asrc_hbm, adst_hbm, m_hbm, h_hbm, src_hbm, dst_hbm, out_hbm,
             asrc_v, adst_v, m_v, srcb, dstb, pbuf, rows, acc_sh, sem):
    cid = lax.axis_index("c")
    sid = lax.axis_index("s")

    pltpu.sync_copy(asrc_hbm, asrc_v)
    pltpu.sync_copy(adst_hbm, adst_v)
    pltpu.sync_copy(m_hbm, m_v)
    mvec = m_v[...]

    @pl.loop(0, B)
    def _zr(e):
        for j in range(D // L):
            rows[e, pl.ds(j * L, L)] = jnp.zeros((L,), jnp.float32)

    @pl.loop(sid, NRCHUNK, step=NS)
    def _za(i):
        pltpu.sync_copy(rows, acc_sh.at[pl.ds(i * RB, RB)])

    plsc.subcore_barrier()

    base_e = (cid * NS + sid) * EPT

    @pl.loop(0, NCHUNK)
    def _chunk(k):
        base = base_e + k * B
        pltpu.sync_copy(src_hbm.at[pl.ds(base, B)], srcb)
        pltpu.sync_copy(dst_hbm.at[pl.ds(base, B)], dstb.at[0])
        for i in range(B // L):
            s_i = srcb[pl.ds(i * L, L)]
            d_i = dstb[0, pl.ds(i * L, L)]
            a_s = plsc.load_gather(asrc_v, [s_i])
            a_d = plsc.load_gather(adst_v, [d_i])
            pre = a_s + a_d
            alpha = jnp.where(pre > 0, pre, 0.2 * pre)
            bnd = mvec + a_d
            shift = jnp.where(bnd > 0, bnd, 0.2 * bnd)
            pbuf[pl.ds(i * L, L)] = jnp.exp(alpha - shift)
        pltpu.async_copy(h_hbm.at[srcb], rows, sem).wait()

        @pl.loop(0, B)
        def _scale(e):
            pv = plsc.load_gather(pbuf, [jnp.broadcast_to(e, (L,))])
            for j in range(D // L):
                rows[e, pl.ds(j * L, L)] = rows[e, pl.ds(j * L, L)] * pv

        pltpu.sync_copy(rows, acc_sh.at[dstb.at[0]], add=True)

    plsc.subcore_barrier()

    @pl.loop(sid, NRCHUNK, step=NS)
    def _rb(i):
        pltpu.sync_copy(acc_sh.at[pl.ds(i * RB, RB)], rows)
        pltpu.sync_copy(rows, out_hbm.at[cid, pl.ds(i * RB, RB)])


def _sc_edge(asrc, adst, mvec, h_aug, src, dst):
    mesh = plsc.VectorSubcoreMesh(core_axis_name="c", subcore_axis_name="s",
                                  num_cores=NC, num_subcores=NS)
    kern = pl.kernel(
        _sc_body,
        out_type=jax.ShapeDtypeStruct((NC, N, D), jnp.float32),
        mesh=mesh,
        scratch_types=[
            pltpu.VMEM((N,), jnp.float32),
            pltpu.VMEM((N,), jnp.float32),
            pltpu.VMEM((L,), jnp.float32),
            pltpu.VMEM((B,), jnp.int32),
            pltpu.VMEM((1, B), jnp.int32),
            pltpu.VMEM((B,), jnp.float32),
            pltpu.VMEM((B, D), jnp.float32),
            pltpu.VMEM_SHARED((N, D), jnp.float32),
            pltpu.SemaphoreType.DMA,
        ],
    )
    return kern(asrc, adst, mvec, h_aug, src, dst)


def _layer(x, Wp, attc, bp, src, dst, d0):
    h, a, m = _dense(x, Wp, attc, d0)
    mvec = jnp.broadcast_to(m.reshape(()), (L,))
    parts = _sc_edge(a[:, 0], a[:, 1], mvec, h, src, dst)
    return _finalize(parts[0], parts[1], h, a, m, bp, d0)


def kernel(x, edge_index, W1, att_src1, att_dst1, b1, W2, att_src2, att_dst2, b2):
    f32 = jnp.float32
    src = edge_index[0].astype(jnp.int32)
    dst = edge_index[1].astype(jnp.int32)

    Wp1 = jnp.zeros((IN_CH_, D), f32).at[:, :HID_].set(W1)
    attc1 = (jnp.zeros((D, 2), f32)
             .at[:HID_, 0].set(att_src1)
             .at[:HID_, 1].set(att_dst1))
    b1p = jnp.zeros((1, D), f32).at[0, :HID_].set(b1)

    Wp2 = jnp.zeros((D, D), f32).at[:HID_, :OUT_].set(W2)
    attc2 = (jnp.zeros((D, 2), f32)
             .at[:OUT_, 0].set(att_src2)
             .at[:OUT_, 1].set(att_dst2))
    b2p = jnp.zeros((1, D), f32).at[0, :OUT_].set(b2)

    g1 = _layer(x, Wp1, attc1, b1p, src, dst, HID_)
    g2 = _layer(g1, Wp2, attc2, b2p, src, dst, OUT_)
    return (g2[:, :OUT_], edge_index)


# speedup vs baseline: 25.9894x; 25.9894x over previous
import functools

import jax
import jax.numpy as jnp
from jax import lax
from jax.experimental import pallas as pl
from jax.experimental.pallas import tpu as pltpu
from jax.experimental.pallas import tpu_sc as plsc

N = 10000
E = 320000
IN_CH_ = 128
HID_ = 50
OUT_ = 40
D = 64
L = 16
NC, NS = 2, 16
NW = NC * NS
EPT = E // NW
B = 80
NCHUNK = EPT // B
RB = 80
NRCHUNK = N // RB


def _dense_body(d0, x_ref, w_ref, attc_ref, h_ref, a_ref, m_ref):
    h = jnp.dot(x_ref[...], w_ref[...], preferred_element_type=jnp.float32,
                precision=lax.Precision.HIGHEST)
    col = lax.broadcasted_iota(jnp.int32, h.shape, 1)
    h = h + jnp.where(col == d0, 1.0, 0.0).astype(jnp.float32)
    h_ref[...] = h
    a = jnp.dot(h, attc_ref[...], preferred_element_type=jnp.float32,
                precision=lax.Precision.HIGHEST)
    a_ref[...] = a
    col2 = lax.broadcasted_iota(jnp.int32, a.shape, 1)
    a_masked = jnp.where(col2 == 0, a, -jnp.inf)
    m_ref[...] = jnp.max(a_masked)[None, None]


def _dense(x, Wp, attc, d0):
    return pl.pallas_call(
        functools.partial(_dense_body, d0),
        out_shape=[
            jax.ShapeDtypeStruct((N, D), jnp.float32),
            jax.ShapeDtypeStruct((N, 2), jnp.float32),
            jax.ShapeDtypeStruct((1, 1), jnp.float32),
        ],
    )(x, Wp, attc)


def _finalize_body(d0, n0_ref, n1_ref, h_ref, a_ref, m_ref, b_ref, g_ref):
    a = a_ref[...]
    col2 = lax.broadcasted_iota(jnp.int32, a.shape, 1)
    a_s = jnp.sum(jnp.where(col2 == 0, a, 0.0), axis=1, keepdims=True)
    a_d = jnp.sum(jnp.where(col2 == 1, a, 0.0), axis=1, keepdims=True)
    m = m_ref[0, 0]
    pre = a_s + a_d
    alpha = jnp.where(pre > 0, pre, 0.2 * pre)
    bnd = m + a_d
    shift = jnp.where(bnd > 0, bnd, 0.2 * bnd)
    p = jnp.exp(alpha - shift)
    num = n0_ref[...] + n1_ref[...] + p * h_ref[...]
    colD = lax.broadcasted_iota(jnp.int32, num.shape, 1)
    den = jnp.sum(jnp.where(colD == d0, num, 0.0), axis=1, keepdims=True)
    g = num / (den + 1e-16) + b_ref[...]
    g_ref[...] = jnp.maximum(g, 0.0)


def _finalize(n0, n1, h, a, m, bp, d0):
    return pl.pallas_call(
        functools.partial(_finalize_body, d0),
        out_shape=jax.ShapeDtypeStruct((N, D), jnp.float32),
    )(n0, n1, h, a, m, bp)


def _sc_body(asrc_hbm, adst_hbm, m_hbm, h_hbm, src_hbm, dst_hbm, out_hbm,
             asrc_v, adst_v, m_v, srcb, dstb, pbuf, rows, acc_sh, sem):
    cid = lax.axis_index("c")
    sid = lax.axis_index("s")

    pltpu.sync_copy(asrc_hbm, asrc_v)
    pltpu.sync_copy(adst_hbm, adst_v)
    pltpu.sync_copy(m_hbm, m_v)
    mvec = m_v[...]

    @pl.loop(0, B)
    def _zr(e):
        for j in range(D // L):
            rows[e, pl.ds(j * L, L)] = jnp.zeros((L,), jnp.float32)

    @pl.loop(sid, NRCHUNK, step=NS)
    def _za(i):
        pltpu.sync_copy(rows, acc_sh.at[pl.ds(i * RB, RB)])

    plsc.subcore_barrier()

    base_e = (cid * NS + sid) * EPT

    @pl.loop(0, NCHUNK)
    def _chunk(k):
        base = base_e + k * B
        pltpu.sync_copy(src_hbm.at[pl.ds(base, B)], srcb)
        pltpu.sync_copy(dst_hbm.at[pl.ds(base, B)], dstb.at[0])
        for i in range(B // L):
            s_i = srcb[pl.ds(i * L, L)]
            d_i = dstb[0, pl.ds(i * L, L)]
            a_s = plsc.load_gather(asrc_v, [s_i])
            a_d = plsc.load_gather(adst_v, [d_i])
            pre = a_s + a_d
            alpha = jnp.where(pre > 0, pre, 0.2 * pre)
            bnd = mvec + a_d
            shift = jnp.where(bnd > 0, bnd, 0.2 * bnd)
            pbuf[pl.ds(i * L, L)] = jnp.exp(alpha - shift)
        pltpu.async_copy(h_hbm.at[srcb], rows, sem).wait()

        @pl.loop(0, B)
        def _scale(e):
            pv = plsc.load_gather(pbuf, [jnp.broadcast_to(e, (L,))])
            for j in range(D // L):
                rows[e, pl.ds(j * L, L)] = rows[e, pl.ds(j * L, L)] * pv

        pltpu.sync_copy(rows, acc_sh.at[dstb.at[0]], add=True)

    plsc.subcore_barrier()

    @pl.loop(sid, NRCHUNK, step=NS)
    def _rb(i):
        pltpu.sync_copy(acc_sh.at[pl.ds(i * RB, RB)], rows)
        pltpu.sync_copy(rows, out_hbm.at[cid, pl.ds(i * RB, RB)])


def _sc_edge(asrc, adst, mvec, h_aug, src, dst):
    mesh = plsc.VectorSubcoreMesh(core_axis_name="c", subcore_axis_name="s",
                                  num_cores=NC, num_subcores=NS)
    kern = pl.kernel(
        _sc_body,
        out_type=jax.ShapeDtypeStruct((NC, N, D), jnp.float32),
        mesh=mesh,
        scratch_types=[
            pltpu.VMEM((N,), jnp.float32),
            pltpu.VMEM((N,), jnp.float32),
            pltpu.VMEM((L,), jnp.float32),
            pltpu.VMEM((B,), jnp.int32),
            pltpu.VMEM((1, B), jnp.int32),
            pltpu.VMEM((B,), jnp.float32),
            pltpu.VMEM((B, D), jnp.float32),
            pltpu.VMEM_SHARED((N, D), jnp.float32),
            pltpu.SemaphoreType.DMA,
        ],
        compiler_params=pltpu.CompilerParams(needs_layout_passes=False,
                                             use_tc_tiling_on_sc=False),
    )
    return kern(asrc, adst, mvec, h_aug, src, dst)


def _layer(x, Wp, attc, bp, src, dst, d0):
    h, a, m = _dense(x, Wp, attc, d0)
    mvec = jnp.broadcast_to(m.reshape(()), (L,))
    parts = _sc_edge(a[:, 0], a[:, 1], mvec, h, src, dst)
    return _finalize(parts[0], parts[1], h, a, m, bp, d0)


def kernel(x, edge_index, W1, att_src1, att_dst1, b1, W2, att_src2, att_dst2, b2):
    f32 = jnp.float32
    src = edge_index[0].astype(jnp.int32)
    dst = edge_index[1].astype(jnp.int32)

    Wp1 = jnp.zeros((IN_CH_, D), f32).at[:, :HID_].set(W1)
    attc1 = (jnp.zeros((D, 2), f32)
             .at[:HID_, 0].set(att_src1)
             .at[:HID_, 1].set(att_dst1))
    b1p = jnp.zeros((1, D), f32).at[0, :HID_].set(b1)

    Wp2 = jnp.zeros((D, D), f32).at[:HID_, :OUT_].set(W2)
    attc2 = (jnp.zeros((D, 2), f32)
             .at[:OUT_, 0].set(att_src2)
             .at[:OUT_, 1].set(att_dst2))
    b2p = jnp.zeros((1, D), f32).at[0, :OUT_].set(b2)

    g1 = _layer(x, Wp1, attc1, b1p, src, dst, HID_)
    g2 = _layer(g1, Wp2, attc2, b2p, src, dst, OUT_)
    return (g2[:, :OUT_], edge_index)


# speedup vs baseline: 51.1910x; 1.9697x over previous
import functools

import jax
import jax.numpy as jnp
from jax import lax
from jax.experimental import pallas as pl
from jax.experimental.pallas import tpu as pltpu
from jax.experimental.pallas import tpu_sc as plsc

N = 10000
E = 320000
IN_CH_ = 128
HID_ = 50
OUT_ = 40
D = 64
L = 16
NC, NS = 2, 16
NW = NC * NS
EPT = E // NW
B = 80
NCHUNK = EPT // B
RB = 80
NRCHUNK = N // RB


def _dense_body(d0, x_ref, w_ref, attc_ref, h_ref, a_ref, m_ref):
    h = jnp.dot(x_ref[...], w_ref[...], preferred_element_type=jnp.float32,
                precision=lax.Precision.HIGHEST)
    col = lax.broadcasted_iota(jnp.int32, h.shape, 1)
    h = h + jnp.where(col == d0, 1.0, 0.0).astype(jnp.float32)
    h_ref[...] = h
    a = jnp.dot(h, attc_ref[...], preferred_element_type=jnp.float32,
                precision=lax.Precision.HIGHEST)
    a_ref[...] = a
    col2 = lax.broadcasted_iota(jnp.int32, a.shape, 1)
    a_masked = jnp.where(col2 == 0, a, -jnp.inf)
    m_ref[...] = jnp.max(a_masked)[None, None]


def _dense(x, Wp, attc, d0):
    return pl.pallas_call(
        functools.partial(_dense_body, d0),
        out_shape=[
            jax.ShapeDtypeStruct((N, D), jnp.float32),
            jax.ShapeDtypeStruct((N, 2), jnp.float32),
            jax.ShapeDtypeStruct((1, 1), jnp.float32),
        ],
    )(x, Wp, attc)


def _finalize_body(d0, n0_ref, n1_ref, h_ref, a_ref, m_ref, b_ref, g_ref):
    a = a_ref[...]
    col2 = lax.broadcasted_iota(jnp.int32, a.shape, 1)
    a_s = jnp.sum(jnp.where(col2 == 0, a, 0.0), axis=1, keepdims=True)
    a_d = jnp.sum(jnp.where(col2 == 1, a, 0.0), axis=1, keepdims=True)
    m = m_ref[0, 0]
    pre = a_s + a_d
    alpha = jnp.where(pre > 0, pre, 0.2 * pre)
    bnd = m + a_d
    shift = jnp.where(bnd > 0, bnd, 0.2 * bnd)
    p = jnp.exp(alpha - shift)
    num = n0_ref[...] + n1_ref[...] + p * h_ref[...]
    colD = lax.broadcasted_iota(jnp.int32, num.shape, 1)
    den = jnp.sum(jnp.where(colD == d0, num, 0.0), axis=1, keepdims=True)
    g = num / (den + 1e-16) + b_ref[...]
    g_ref[...] = jnp.maximum(g, 0.0)


def _finalize(n0, n1, h, a, m, bp, d0):
    return pl.pallas_call(
        functools.partial(_finalize_body, d0),
        out_shape=jax.ShapeDtypeStruct((N, D), jnp.float32),
    )(n0, n1, h, a, m, bp)


def _sc_body(asrc_hbm, adst_hbm, m_hbm, h_hbm, src_hbm, dst_hbm, out_hbm,
             asrc_v, adst_v, m_v, src_all, dst_all, pall, rows0, rows1,
             acc_sh, sem0, sem1):
    cid = lax.axis_index("c")
    sid = lax.axis_index("s")
    w = cid * NS + sid

    pltpu.sync_copy(asrc_hbm, asrc_v)
    pltpu.sync_copy(adst_hbm, adst_v)
    pltpu.sync_copy(m_hbm, m_v)
    pltpu.sync_copy(src_hbm.at[pl.ds(w * NCHUNK, NCHUNK)], src_all)
    pltpu.sync_copy(dst_hbm.at[pl.ds(w * NCHUNK, NCHUNK)], dst_all)
    mvec = m_v[...]

    @pl.loop(0, B)
    def _zr(e):
        for j in range(D // L):
            rows0[e, pl.ds(j * L, L)] = jnp.zeros((L,), jnp.float32)

    @pl.loop(sid, NRCHUNK, step=NS)
    def _za(i):
        pltpu.sync_copy(rows0, acc_sh.at[pl.ds(i * RB, RB)])

    @pl.loop(0, NCHUNK)
    def _pp(k):
        for i in range(B // L):
            s_i = src_all[k, pl.ds(i * L, L)]
            d_i = dst_all[k, pl.ds(i * L, L)]
            a_s = plsc.load_gather(asrc_v, [s_i])
            a_d = plsc.load_gather(adst_v, [d_i])
            pre = a_s + a_d
            alpha = jnp.where(pre > 0, pre, 0.2 * pre)
            bnd = mvec + a_d
            shift = jnp.where(bnd > 0, bnd, 0.2 * bnd)
            pall[pl.ds(k * B + i * L, L)] = jnp.exp(alpha - shift)

    plsc.subcore_barrier()

    bufs = ((rows0, sem0), (rows1, sem1))

    def _issue(kk, rows, sem):
        pltpu.async_copy(h_hbm.at[src_all.at[kk]], rows, sem)

    def _wait(kk, rows, sem):
        pltpu.make_async_copy(h_hbm.at[src_all.at[kk]], rows, sem).wait()

    _issue(0, rows0, sem0)
    _issue(1, rows1, sem1)

    @pl.loop(0, NCHUNK, step=2)
    def _chunk(k):
        for b in range(2):
            kk = k + b
            rows, sem = bufs[b]

            @pl.when(kk < NCHUNK)
            def _do():
                _wait(kk, rows, sem)

                @pl.loop(0, B, unroll=8)
                def _scale(e):
                    pv = plsc.load_gather(
                        pall, [jnp.broadcast_to(kk * B + e, (L,))])
                    for j in range(D // L):
                        rows[e, pl.ds(j * L, L)] = rows[e, pl.ds(j * L, L)] * pv

                pltpu.sync_copy(rows, acc_sh.at[dst_all.at[kk]], add=True)

                @pl.when(kk + 2 < NCHUNK)
                def _pf():
                    _issue(kk + 2, rows, sem)

    plsc.subcore_barrier()

    @pl.loop(sid, NRCHUNK, step=NS)
    def _rb(i):
        pltpu.sync_copy(acc_sh.at[pl.ds(i * RB, RB)], rows0)
        pltpu.sync_copy(rows0, out_hbm.at[cid, pl.ds(i * RB, RB)])


def _sc_edge(asrc, adst, mvec, h_aug, src, dst):
    mesh = plsc.VectorSubcoreMesh(core_axis_name="c", subcore_axis_name="s",
                                  num_cores=NC, num_subcores=NS)
    kern = pl.kernel(
        _sc_body,
        out_type=jax.ShapeDtypeStruct((NC, N, D), jnp.float32),
        mesh=mesh,
        scratch_types=[
            pltpu.VMEM((N,), jnp.float32),
            pltpu.VMEM((N,), jnp.float32),
            pltpu.VMEM((L,), jnp.float32),
            pltpu.VMEM((NCHUNK, B), jnp.int32),
            pltpu.VMEM((NCHUNK, B), jnp.int32),
            pltpu.VMEM((EPT,), jnp.float32),
            pltpu.VMEM((B, D), jnp.float32),
            pltpu.VMEM((B, D), jnp.float32),
            pltpu.VMEM_SHARED((N, D), jnp.float32),
            pltpu.SemaphoreType.DMA,
            pltpu.SemaphoreType.DMA,
        ],
        compiler_params=pltpu.CompilerParams(needs_layout_passes=False,
                                             use_tc_tiling_on_sc=False),
    )
    return kern(asrc, adst, mvec, h_aug, src, dst)


def _layer(x, Wp, attc, bp, src, dst, d0):
    h, a, m = _dense(x, Wp, attc, d0)
    mvec = jnp.broadcast_to(m.reshape(()), (L,))
    parts = _sc_edge(a[:, 0], a[:, 1], mvec, h,
                     src.reshape(E // B, B), dst.reshape(E // B, B))
    return _finalize(parts[0], parts[1], h, a, m, bp, d0)


def kernel(x, edge_index, W1, att_src1, att_dst1, b1, W2, att_src2, att_dst2, b2):
    f32 = jnp.float32
    src = edge_index[0].astype(jnp.int32)
    dst = edge_index[1].astype(jnp.int32)

    Wp1 = jnp.zeros((IN_CH_, D), f32).at[:, :HID_].set(W1)
    attc1 = (jnp.zeros((D, 2), f32)
             .at[:HID_, 0].set(att_src1)
             .at[:HID_, 1].set(att_dst1))
    b1p = jnp.zeros((1, D), f32).at[0, :HID_].set(b1)

    Wp2 = jnp.zeros((D, D), f32).at[:HID_, :OUT_].set(W2)
    attc2 = (jnp.zeros((D, 2), f32)
             .at[:OUT_, 0].set(att_src2)
             .at[:OUT_, 1].set(att_dst2))
    b2p = jnp.zeros((1, D), f32).at[0, :OUT_].set(b2)

    g1 = _layer(x, Wp1, attc1, b1p, src, dst, HID_)
    g2 = _layer(g1, Wp2, attc2, b2p, src, dst, OUT_)
    return (g2[:, :OUT_], edge_index)


# speedup vs baseline: 55.1108x; 1.0766x over previous
import functools

import jax
import jax.numpy as jnp
from jax import lax
from jax.experimental import pallas as pl
from jax.experimental.pallas import tpu as pltpu
from jax.experimental.pallas import tpu_sc as plsc

N = 10000
E = 320000
IN_CH_ = 128
HID_ = 50
OUT_ = 40
D = 64
L = 16
NC, NS = 2, 16
NW = NC * NS
EPT = E // NW
B = 80
NCHUNK = EPT // B
RB = 80
NRCHUNK = N // RB


def _dense_body(d0, x_ref, w_ref, attc_ref, h_ref, a_ref, m_ref):
    h = jnp.dot(x_ref[...], w_ref[...], preferred_element_type=jnp.float32,
                precision=lax.Precision.HIGHEST)
    col = lax.broadcasted_iota(jnp.int32, h.shape, 1)
    h = h + jnp.where(col == d0, 1.0, 0.0).astype(jnp.float32)
    h_ref[...] = h
    a = jnp.dot(h, attc_ref[...], preferred_element_type=jnp.float32,
                precision=lax.Precision.HIGHEST)
    a_ref[...] = a
    col2 = lax.broadcasted_iota(jnp.int32, a.shape, 1)
    a_masked = jnp.where(col2 == 0, a, -jnp.inf)
    m_ref[...] = jnp.max(a_masked)[None, None]


def _dense(x, Wp, attc, d0):
    return pl.pallas_call(
        functools.partial(_dense_body, d0),
        out_shape=[
            jax.ShapeDtypeStruct((N, D), jnp.float32),
            jax.ShapeDtypeStruct((N, 2), jnp.float32),
            jax.ShapeDtypeStruct((1, 1), jnp.float32),
        ],
    )(x, Wp, attc)


def _finalize_body(d0, n0_ref, n1_ref, h_ref, a_ref, m_ref, b_ref, g_ref):
    a = a_ref[...]
    col2 = lax.broadcasted_iota(jnp.int32, a.shape, 1)
    a_s = jnp.sum(jnp.where(col2 == 0, a, 0.0), axis=1, keepdims=True)
    a_d = jnp.sum(jnp.where(col2 == 1, a, 0.0), axis=1, keepdims=True)
    m = m_ref[0, 0]
    pre = a_s + a_d
    alpha = jnp.where(pre > 0, pre, 0.2 * pre)
    bnd = m + a_d
    shift = jnp.where(bnd > 0, bnd, 0.2 * bnd)
    p = jnp.exp(alpha - shift)
    num = n0_ref[...] + n1_ref[...] + p * h_ref[...]
    colD = lax.broadcasted_iota(jnp.int32, num.shape, 1)
    den = jnp.sum(jnp.where(colD == d0, num, 0.0), axis=1, keepdims=True)
    g = num / (den + 1e-16) + b_ref[...]
    g_ref[...] = jnp.maximum(g, 0.0)


def _finalize(n0, n1, h, a, m, bp, d0):
    return pl.pallas_call(
        functools.partial(_finalize_body, d0),
        out_shape=jax.ShapeDtypeStruct((N, D), jnp.float32),
    )(n0, n1, h, a, m, bp)


def _sc_body(asrc_hbm, adst_hbm, m_hbm, h_hbm, src_hbm, dst_hbm, out_hbm,
             asrc_v, adst_v, m_v, src_all, dst_all, pall,
             rows0, rows1, rows2, rows3, acc_sh,
             gsem0, gsem1, gsem2, gsem3, ssem0, ssem1, ssem2, ssem3):
    cid = lax.axis_index("c")
    sid = lax.axis_index("s")
    w = cid * NS + sid

    pltpu.sync_copy(asrc_hbm, asrc_v)
    pltpu.sync_copy(adst_hbm, adst_v)
    pltpu.sync_copy(m_hbm, m_v)
    pltpu.sync_copy(src_hbm.at[pl.ds(w * NCHUNK, NCHUNK)], src_all)
    pltpu.sync_copy(dst_hbm.at[pl.ds(w * NCHUNK, NCHUNK)], dst_all)
    mvec = m_v[...]

    @pl.loop(0, B)
    def _zr(e):
        for j in range(D // L):
            rows0[e, pl.ds(j * L, L)] = jnp.zeros((L,), jnp.float32)

    @pl.loop(sid, NRCHUNK, step=NS)
    def _za(i):
        pltpu.sync_copy(rows0, acc_sh.at[pl.ds(i * RB, RB)])

    @pl.loop(0, NCHUNK)
    def _pp(k):
        for i in range(B // L):
            s_i = src_all[k, pl.ds(i * L, L)]
            d_i = dst_all[k, pl.ds(i * L, L)]
            a_s = plsc.load_gather(asrc_v, [s_i])
            a_d = plsc.load_gather(adst_v, [d_i])
            pre = a_s + a_d
            alpha = jnp.where(pre > 0, pre, 0.2 * pre)
            bnd = mvec + a_d
            shift = jnp.where(bnd > 0, bnd, 0.2 * bnd)
            pall[pl.ds(k * B + i * L, L)] = jnp.exp(alpha - shift)

    plsc.subcore_barrier()

    bufs = (rows0, rows1, rows2, rows3)
    gsems = (gsem0, gsem1, gsem2, gsem3)
    ssems = (ssem0, ssem1, ssem2, ssem3)
    NBUF = 4

    def _g_issue(kk, b):
        pltpu.async_copy(h_hbm.at[src_all.at[kk]], bufs[b], gsems[b])

    def _g_wait(kk, b):
        pltpu.make_async_copy(h_hbm.at[src_all.at[kk]], bufs[b],
                              gsems[b]).wait()

    def _s_issue(kk, b):
        pltpu.async_copy(bufs[b], acc_sh.at[dst_all.at[kk]], ssems[b],
                         add=True)

    def _s_wait(kk, b):
        pltpu.make_async_copy(bufs[b], acc_sh.at[dst_all.at[kk]],
                              ssems[b]).wait()

    for b in range(NBUF):
        _g_issue(b, b)

    @pl.loop(0, NCHUNK, step=NBUF)
    def _chunk(k):
        for b in range(NBUF):
            kk = k + b
            rows = bufs[b]

            @pl.when(kk < NCHUNK)
            def _do():
                _g_wait(kk, b)

                @pl.loop(0, B, unroll=8)
                def _scale(e):
                    pv = plsc.load_gather(
                        pall, [jnp.broadcast_to(kk * B + e, (L,))])
                    for j in range(D // L):
                        rows[e, pl.ds(j * L, L)] = rows[e, pl.ds(j * L, L)] * pv

                _s_issue(kk, b)

                @pl.when(kk >= 2)
                def _pf():
                    _s_wait(kk - 2, (b + 2) % NBUF)

                    @pl.when(kk + 2 < NCHUNK)
                    def _pf2():
                        _g_issue(kk + 2, (b + 2) % NBUF)

    _s_wait(NCHUNK - 2, (NCHUNK - 2) % NBUF)
    _s_wait(NCHUNK - 1, (NCHUNK - 1) % NBUF)

    plsc.subcore_barrier()

    @pl.loop(sid, NRCHUNK, step=NS)
    def _rb(i):
        pltpu.sync_copy(acc_sh.at[pl.ds(i * RB, RB)], rows0)
        pltpu.sync_copy(rows0, out_hbm.at[cid, pl.ds(i * RB, RB)])


def _sc_edge(asrc, adst, mvec, h_aug, src, dst):
    mesh = plsc.VectorSubcoreMesh(core_axis_name="c", subcore_axis_name="s",
                                  num_cores=NC, num_subcores=NS)
    kern = pl.kernel(
        _sc_body,
        out_type=jax.ShapeDtypeStruct((NC, N, D), jnp.float32),
        mesh=mesh,
        scratch_types=[
            pltpu.VMEM((N,), jnp.float32),
            pltpu.VMEM((N,), jnp.float32),
            pltpu.VMEM((L,), jnp.float32),
            pltpu.VMEM((NCHUNK, B), jnp.int32),
            pltpu.VMEM((NCHUNK, B), jnp.int32),
            pltpu.VMEM((EPT,), jnp.float32),
            pltpu.VMEM((B, D), jnp.float32),
            pltpu.VMEM((B, D), jnp.float32),
            pltpu.VMEM((B, D), jnp.float32),
            pltpu.VMEM((B, D), jnp.float32),
            pltpu.VMEM_SHARED((N, D), jnp.float32),
            pltpu.SemaphoreType.DMA,
            pltpu.SemaphoreType.DMA,
            pltpu.SemaphoreType.DMA,
            pltpu.SemaphoreType.DMA,
            pltpu.SemaphoreType.DMA,
            pltpu.SemaphoreType.DMA,
            pltpu.SemaphoreType.DMA,
            pltpu.SemaphoreType.DMA,
        ],
        compiler_params=pltpu.CompilerParams(needs_layout_passes=False,
                                             use_tc_tiling_on_sc=False),
    )
    return kern(asrc, adst, mvec, h_aug, src, dst)


def _layer(x, Wp, attc, bp, src, dst, d0):
    h, a, m = _dense(x, Wp, attc, d0)
    mvec = jnp.broadcast_to(m.reshape(()), (L,))
    parts = _sc_edge(a[:, 0], a[:, 1], mvec, h,
                     src.reshape(E // B, B), dst.reshape(E // B, B))
    return _finalize(parts[0], parts[1], h, a, m, bp, d0)


def kernel(x, edge_index, W1, att_src1, att_dst1, b1, W2, att_src2, att_dst2, b2):
    f32 = jnp.float32
    src = edge_index[0].astype(jnp.int32)
    dst = edge_index[1].astype(jnp.int32)

    Wp1 = jnp.zeros((IN_CH_, D), f32).at[:, :HID_].set(W1)
    attc1 = (jnp.zeros((D, 2), f32)
             .at[:HID_, 0].set(att_src1)
             .at[:HID_, 1].set(att_dst1))
    b1p = jnp.zeros((1, D), f32).at[0, :HID_].set(b1)

    Wp2 = jnp.zeros((D, D), f32).at[:HID_, :OUT_].set(W2)
    attc2 = (jnp.zeros((D, 2), f32)
             .at[:OUT_, 0].set(att_src2)
             .at[:OUT_, 1].set(att_dst2))
    b2p = jnp.zeros((1, D), f32).at[0, :OUT_].set(b2)

    g1 = _layer(x, Wp1, attc1, b1p, src, dst, HID_)
    g2 = _layer(g1, Wp2, attc2, b2p, src, dst, OUT_)
    return (g2[:, :OUT_], edge_index)


# speedup vs baseline: 58.6677x; 1.0645x over previous
import jax
import jax.numpy as jnp
from jax import lax
from jax.experimental import pallas as pl
from jax.experimental.pallas import tpu as pltpu
from jax.experimental.pallas import tpu_sc as plsc

N = 10000
E = 320000
IN_CH_ = 128
HID_ = 50
OUT_ = 40
D = 64
L = 16
NC, NS = 2, 16
NW = NC * NS
EPT = E // NW
B = 80
NCHUNK = EPT // B
RB = 80
NRCHUNK = N // RB


def _attention(h, attc_ref):
    a = jnp.dot(h, attc_ref[...], preferred_element_type=jnp.float32,
                precision=lax.Precision.HIGHEST)
    col2 = lax.broadcasted_iota(jnp.int32, a.shape, 1)
    a_masked = jnp.where(col2 == 0, a, -jnp.inf)
    m = jnp.broadcast_to(jnp.max(a_masked), (1, L))
    return a, m


def _matmul_pad(x, w_ref, d0):
    h = jnp.dot(x, w_ref[...], preferred_element_type=jnp.float32,
                precision=lax.Precision.HIGHEST)
    col = lax.broadcasted_iota(jnp.int32, h.shape, 1)
    return h + jnp.where(col == d0, 1.0, 0.0).astype(jnp.float32)


def _dense1_body(x_ref, w_ref, attc_ref, h_ref, a_ref, m_ref):
    h = _matmul_pad(x_ref[...], w_ref, HID_)
    h_ref[...] = h
    a, m = _attention(h, attc_ref)
    a_ref[...] = a
    m_ref[...] = m


def _dense1(x, Wp, attc):
    return pl.pallas_call(
        _dense1_body,
        out_shape=[
            jax.ShapeDtypeStruct((N, D), jnp.float32),
            jax.ShapeDtypeStruct((N, 2), jnp.float32),
            jax.ShapeDtypeStruct((1, L), jnp.float32),
        ],
    )(x, Wp, attc)


def _gat_finalize(d0, parts_ref, h_ref, a_ref, m_ref, b_ref):
    a = a_ref[...]
    col2 = lax.broadcasted_iota(jnp.int32, a.shape, 1)
    a_s = jnp.sum(jnp.where(col2 == 0, a, 0.0), axis=1, keepdims=True)
    a_d = jnp.sum(jnp.where(col2 == 1, a, 0.0), axis=1, keepdims=True)
    m = m_ref[0, 0]
    pre = a_s + a_d
    alpha = jnp.where(pre > 0, pre, 0.2 * pre)
    bnd = m + a_d
    shift = jnp.where(bnd > 0, bnd, 0.2 * bnd)
    p = jnp.exp(alpha - shift)
    num = parts_ref[0] + parts_ref[1] + p * h_ref[...]
    colD = lax.broadcasted_iota(jnp.int32, num.shape, 1)
    den = jnp.sum(jnp.where(colD == d0, num, 0.0), axis=1, keepdims=True)
    g = num / (den + 1e-16) + b_ref[...]
    return jnp.maximum(g, 0.0)


def _mid_body(parts_ref, h_ref, a_ref, m_ref, b_ref, w2_ref, attc2_ref,
              h2_ref, a2_ref, m2_ref):
    g1 = _gat_finalize(HID_, parts_ref, h_ref, a_ref, m_ref, b_ref)
    h2 = _matmul_pad(g1, w2_ref, OUT_)
    h2_ref[...] = h2
    a2, m2 = _attention(h2, attc2_ref)
    a2_ref[...] = a2
    m2_ref[...] = m2


def _mid(parts, h1, a1, m1, b1p, Wp2, attc2):
    return pl.pallas_call(
        _mid_body,
        out_shape=[
            jax.ShapeDtypeStruct((N, D), jnp.float32),
            jax.ShapeDtypeStruct((N, 2), jnp.float32),
            jax.ShapeDtypeStruct((1, L), jnp.float32),
        ],
    )(parts, h1, a1, m1, b1p, Wp2, attc2)


def _fin2_body(parts_ref, h_ref, a_ref, m_ref, b_ref, g_ref):
    g_ref[...] = _gat_finalize(OUT_, parts_ref, h_ref, a_ref, m_ref, b_ref)


def _fin2(parts, h2, a2, m2, b2p):
    return pl.pallas_call(
        _fin2_body,
        out_shape=jax.ShapeDtypeStruct((N, D), jnp.float32),
    )(parts, h2, a2, m2, b2p)


def _sc_body(asrc_hbm, adst_hbm, m_hbm, h_hbm, src_hbm, dst_hbm, out_hbm,
             asrc_v, adst_v, m_v, src_all, dst_all, pall,
             rows0, rows1, rows2, rows3, acc_sh,
             gsem0, gsem1, gsem2, gsem3, ssem0, ssem1, ssem2, ssem3):
    cid = lax.axis_index("c")
    sid = lax.axis_index("s")
    w = cid * NS + sid

    pltpu.sync_copy(asrc_hbm, asrc_v)
    pltpu.sync_copy(adst_hbm, adst_v)
    pltpu.sync_copy(m_hbm, m_v)
    pltpu.sync_copy(src_hbm.at[pl.ds(w * NCHUNK, NCHUNK)], src_all)
    pltpu.sync_copy(dst_hbm.at[pl.ds(w * NCHUNK, NCHUNK)], dst_all)
    mvec = m_v[...]

    @pl.loop(0, B)
    def _zr(e):
        for j in range(D // L):
            rows0[e, pl.ds(j * L, L)] = jnp.zeros((L,), jnp.float32)

    @pl.loop(sid, NRCHUNK, step=NS)
    def _za(i):
        pltpu.sync_copy(rows0, acc_sh.at[pl.ds(i * RB, RB)])

    @pl.loop(0, NCHUNK)
    def _pp(k):
        for i in range(B // L):
            s_i = src_all[k, pl.ds(i * L, L)]
            d_i = dst_all[k, pl.ds(i * L, L)]
            a_s = plsc.load_gather(asrc_v, [s_i])
            a_d = plsc.load_gather(adst_v, [d_i])
            pre = a_s + a_d
            alpha = jnp.where(pre > 0, pre, 0.2 * pre)
            bnd = mvec + a_d
            shift = jnp.where(bnd > 0, bnd, 0.2 * bnd)
            pall[pl.ds(k * B + i * L, L)] = jnp.exp(alpha - shift)

    plsc.subcore_barrier()

    bufs = (rows0, rows1, rows2, rows3)
    gsems = (gsem0, gsem1, gsem2, gsem3)
    ssems = (ssem0, ssem1, ssem2, ssem3)
    NBUF = 4

    def _g_issue(kk, b):
        pltpu.async_copy(h_hbm.at[src_all.at[kk]], bufs[b], gsems[b])

    def _g_wait(kk, b):
        pltpu.make_async_copy(h_hbm.at[src_all.at[kk]], bufs[b],
                              gsems[b]).wait()

    def _s_issue(kk, b):
        pltpu.async_copy(bufs[b], acc_sh.at[dst_all.at[kk]], ssems[b],
                         add=True)

    def _s_wait(kk, b):
        pltpu.make_async_copy(bufs[b], acc_sh.at[dst_all.at[kk]],
                              ssems[b]).wait()

    for b in range(NBUF):
        _g_issue(b, b)

    @pl.loop(0, NCHUNK, step=NBUF)
    def _chunk(k):
        for b in range(NBUF):
            kk = k + b
            rows = bufs[b]

            @pl.when(kk < NCHUNK)
            def _do():
                _g_wait(kk, b)

                @pl.loop(0, B, unroll=8)
                def _scale(e):
                    pv = plsc.load_gather(
                        pall, [jnp.broadcast_to(kk * B + e, (L,))])
                    for j in range(D // L):
                        rows[e, pl.ds(j * L, L)] = rows[e, pl.ds(j * L, L)] * pv

                _s_issue(kk, b)

                @pl.when(kk >= 2)
                def _pf():
                    _s_wait(kk - 2, (b + 2) % NBUF)

                    @pl.when(kk + 2 < NCHUNK)
                    def _pf2():
                        _g_issue(kk + 2, (b + 2) % NBUF)

    _s_wait(NCHUNK - 2, (NCHUNK - 2) % NBUF)
    _s_wait(NCHUNK - 1, (NCHUNK - 1) % NBUF)

    plsc.subcore_barrier()

    @pl.loop(sid, NRCHUNK, step=NS)
    def _rb(i):
        pltpu.sync_copy(acc_sh.at[pl.ds(i * RB, RB)], rows0)
        pltpu.sync_copy(rows0, out_hbm.at[cid, pl.ds(i * RB, RB)])


def _sc_edge(asrc, adst, m, h_aug, src2, dst2):
    mesh = plsc.VectorSubcoreMesh(core_axis_name="c", subcore_axis_name="s",
                                  num_cores=NC, num_subcores=NS)
    kern = pl.kernel(
        _sc_body,
        out_type=jax.ShapeDtypeStruct((NC, N, D), jnp.float32),
        mesh=mesh,
        scratch_types=[
            pltpu.VMEM((N,), jnp.float32),
            pltpu.VMEM((N,), jnp.float32),
            pltpu.VMEM((L,), jnp.float32),
            pltpu.VMEM((NCHUNK, B), jnp.int32),
            pltpu.VMEM((NCHUNK, B), jnp.int32),
            pltpu.VMEM((EPT,), jnp.float32),
            pltpu.VMEM((B, D), jnp.float32),
            pltpu.VMEM((B, D), jnp.float32),
            pltpu.VMEM((B, D), jnp.float32),
            pltpu.VMEM((B, D), jnp.float32),
            pltpu.VMEM_SHARED((N, D), jnp.float32),
            pltpu.SemaphoreType.DMA,
            pltpu.SemaphoreType.DMA,
            pltpu.SemaphoreType.DMA,
            pltpu.SemaphoreType.DMA,
            pltpu.SemaphoreType.DMA,
            pltpu.SemaphoreType.DMA,
            pltpu.SemaphoreType.DMA,
            pltpu.SemaphoreType.DMA,
        ],
        compiler_params=pltpu.CompilerParams(needs_layout_passes=False,
                                             use_tc_tiling_on_sc=False),
    )
    return kern(asrc, adst, m, h_aug, src2, dst2)


def kernel(x, edge_index, W1, att_src1, att_dst1, b1, W2, att_src2, att_dst2, b2):
    f32 = jnp.float32
    src2 = edge_index[0].astype(jnp.int32).reshape(E // B, B)
    dst2 = edge_index[1].astype(jnp.int32).reshape(E // B, B)

    Wp1 = jnp.zeros((IN_CH_, D), f32).at[:, :HID_].set(W1)
    attc1 = (jnp.zeros((D, 2), f32)
             .at[:HID_, 0].set(att_src1)
             .at[:HID_, 1].set(att_dst1))
    b1p = jnp.zeros((1, D), f32).at[0, :HID_].set(b1)

    Wp2 = jnp.zeros((D, D), f32).at[:HID_, :OUT_].set(W2)
    attc2 = (jnp.zeros((D, 2), f32)
             .at[:OUT_, 0].set(att_src2)
             .at[:OUT_, 1].set(att_dst2))
    b2p = jnp.zeros((1, D), f32).at[0, :OUT_].set(b2)

    h1, a1, m1 = _dense1(x, Wp1, attc1)
    parts1 = _sc_edge(a1[:, 0], a1[:, 1], m1.reshape(L), h1, src2, dst2)
    h2, a2, m2 = _mid(parts1, h1, a1, m1, b1p, Wp2, attc2)
    parts2 = _sc_edge(a2[:, 0], a2[:, 1], m2.reshape(L), h2, src2, dst2)
    g2 = _fin2(parts2, h2, a2, m2, b2p)
    return (g2[:, :OUT_], edge_index)


# speedup vs baseline: 60.6742x; 1.0342x over previous
import jax
import jax.numpy as jnp
from jax import lax
from jax.experimental import pallas as pl
from jax.experimental.pallas import tpu as pltpu
from jax.experimental.pallas import tpu_sc as plsc

N = 10000
E = 320000
IN_CH_ = 128
HID_ = 50
OUT_ = 40
D = 64
L = 16
NC, NS = 2, 16
NW = NC * NS
EPT = E // NW
B = 80
NCHUNK = EPT // B
ZB = 125


def _attention(h, attc_ref):
    a = jnp.dot(h, attc_ref[...], preferred_element_type=jnp.float32,
                precision=lax.Precision.HIGHEST)
    col2 = lax.broadcasted_iota(jnp.int32, a.shape, 1)
    a_masked = jnp.where(col2 == 0, a, -jnp.inf)
    m = jnp.broadcast_to(jnp.max(a_masked), (1, L))
    return a, m


def _matmul_pad(x, w_ref, d0):
    h = jnp.dot(x, w_ref[...], preferred_element_type=jnp.float32,
                precision=lax.Precision.HIGHEST)
    col = lax.broadcasted_iota(jnp.int32, h.shape, 1)
    return h + jnp.where(col == d0, 1.0, 0.0).astype(jnp.float32)


def _dense1_body(x_ref, w_ref, attc_ref, h_ref, a_ref, m_ref):
    h = _matmul_pad(x_ref[...], w_ref, HID_)
    h_ref[...] = h
    a, m = _attention(h, attc_ref)
    a_ref[...] = a
    m_ref[...] = m


def _dense1(x, Wp, attc):
    return pl.pallas_call(
        _dense1_body,
        out_shape=[
            jax.ShapeDtypeStruct((N, D), jnp.float32),
            jax.ShapeDtypeStruct((N, 2), jnp.float32),
            jax.ShapeDtypeStruct((1, L), jnp.float32),
        ],
    )(x, Wp, attc)


def _gat_finalize(d0, parts_ref, h_ref, a_ref, m_ref, b_ref):
    a = a_ref[...]
    col2 = lax.broadcasted_iota(jnp.int32, a.shape, 1)
    a_s = jnp.sum(jnp.where(col2 == 0, a, 0.0), axis=1, keepdims=True)
    a_d = jnp.sum(jnp.where(col2 == 1, a, 0.0), axis=1, keepdims=True)
    m = m_ref[0, 0]
    pre = a_s + a_d
    alpha = jnp.where(pre > 0, pre, 0.2 * pre)
    bnd = m + a_d
    shift = jnp.where(bnd > 0, bnd, 0.2 * bnd)
    p = jnp.exp(alpha - shift)
    num = parts_ref[0] + parts_ref[1] + p * h_ref[...]
    colD = lax.broadcasted_iota(jnp.int32, num.shape, 1)
    den = jnp.sum(jnp.where(colD == d0, num, 0.0), axis=1, keepdims=True)
    g = num / (den + 1e-16) + b_ref[...]
    return jnp.maximum(g, 0.0)


def _mid_body(parts_ref, h_ref, a_ref, m_ref, b_ref, w2_ref, attc2_ref,
              h2_ref, a2_ref, m2_ref):
    g1 = _gat_finalize(HID_, parts_ref, h_ref, a_ref, m_ref, b_ref)
    h2 = _matmul_pad(g1, w2_ref, OUT_)
    h2_ref[...] = h2
    a2, m2 = _attention(h2, attc2_ref)
    a2_ref[...] = a2
    m2_ref[...] = m2


def _mid(parts, h1, a1, m1, b1p, Wp2, attc2):
    return pl.pallas_call(
        _mid_body,
        out_shape=[
            jax.ShapeDtypeStruct((N, D), jnp.float32),
            jax.ShapeDtypeStruct((N, 2), jnp.float32),
            jax.ShapeDtypeStruct((1, L), jnp.float32),
        ],
    )(parts, h1, a1, m1, b1p, Wp2, attc2)


def _fin2_body(parts_ref, h_ref, a_ref, m_ref, b_ref, g_ref):
    g_ref[...] = _gat_finalize(OUT_, parts_ref, h_ref, a_ref, m_ref, b_ref)


def _fin2(parts, h2, a2, m2, b2p):
    return pl.pallas_call(
        _fin2_body,
        out_shape=jax.ShapeDtypeStruct((N, D), jnp.float32),
    )(parts, h2, a2, m2, b2p)


def _sc_body(asrc_hbm, adst_hbm, m_hbm, h_hbm, src_hbm, dst_hbm, out_hbm,
             asrc_v, adst_v, m_v, src_all, dst_all, pall,
             rows0, rows1, rows2, rows3, zbuf, rbuf, acc_sh,
             gsem0, gsem1, gsem2, gsem3, ssem0, ssem1, ssem2, ssem3):
    cid = lax.axis_index("c")
    sid = lax.axis_index("s")
    w = cid * NS + sid

    c0 = pltpu.async_copy(asrc_hbm, asrc_v, gsem0)
    c1 = pltpu.async_copy(adst_hbm, adst_v, gsem1)
    c2 = pltpu.async_copy(m_hbm, m_v, gsem2)
    c3 = pltpu.async_copy(src_hbm.at[pl.ds(w * NCHUNK, NCHUNK)], src_all,
                          gsem3)
    c4 = pltpu.async_copy(dst_hbm.at[pl.ds(w * NCHUNK, NCHUNK)], dst_all,
                          ssem0)

    @pl.loop(0, ZB)
    def _zr(e):
        for j in range(D // L):
            zbuf[e, pl.ds(j * L, L)] = jnp.zeros((L,), jnp.float32)

    zc = [pltpu.async_copy(zbuf, acc_sh.at[pl.ds(sid * (N // NS) + j * ZB, ZB)],
                           ssem1) for j in range((N // NS) // ZB)]
    c0.wait(); c1.wait(); c2.wait(); c3.wait(); c4.wait()
    mvec = m_v[...]

    @pl.loop(0, NCHUNK)
    def _pp(k):
        for i in range(B // L):
            s_i = src_all[k, pl.ds(i * L, L)]
            d_i = dst_all[k, pl.ds(i * L, L)]
            a_s = plsc.load_gather(asrc_v, [s_i])
            a_d = plsc.load_gather(adst_v, [d_i])
            pre = a_s + a_d
            alpha = jnp.where(pre > 0, pre, 0.2 * pre)
            bnd = mvec + a_d
            shift = jnp.where(bnd > 0, bnd, 0.2 * bnd)
            pall[pl.ds(k * B + i * L, L)] = jnp.exp(alpha - shift)

    for c in zc:
        c.wait()
    plsc.subcore_barrier()

    bufs = (rows0, rows1, rows2, rows3)
    gsems = (gsem0, gsem1, gsem2, gsem3)
    ssems = (ssem0, ssem1, ssem2, ssem3)
    NBUF = 4

    def _g_issue(kk, b):
        pltpu.async_copy(h_hbm.at[src_all.at[kk]], bufs[b], gsems[b])

    def _g_wait(kk, b):
        pltpu.make_async_copy(h_hbm.at[src_all.at[kk]], bufs[b],
                              gsems[b]).wait()

    def _s_issue(kk, b):
        pltpu.async_copy(bufs[b], acc_sh.at[dst_all.at[kk]], ssems[b],
                         add=True)

    def _s_wait(kk, b):
        pltpu.make_async_copy(bufs[b], acc_sh.at[dst_all.at[kk]],
                              ssems[b]).wait()

    for b in range(NBUF):
        _g_issue(b, b)

    @pl.loop(0, NCHUNK, step=NBUF)
    def _chunk(k):
        for b in range(NBUF):
            kk = k + b
            rows = bufs[b]

            @pl.when(kk < NCHUNK)
            def _do():
                _g_wait(kk, b)

                @pl.loop(0, B, unroll=8)
                def _scale(e):
                    pv = plsc.load_gather(
                        pall, [jnp.broadcast_to(kk * B + e, (L,))])
                    for j in range(D // L):
                        rows[e, pl.ds(j * L, L)] = rows[e, pl.ds(j * L, L)] * pv

                _s_issue(kk, b)

                @pl.when(kk >= 2)
                def _pf():
                    _s_wait(kk - 2, (b + 2) % NBUF)

                    @pl.when(kk + 2 < NCHUNK)
                    def _pf2():
                        _g_issue(kk + 2, (b + 2) % NBUF)

    _s_wait(NCHUNK - 2, (NCHUNK - 2) % NBUF)
    _s_wait(NCHUNK - 1, (NCHUNK - 1) % NBUF)

    plsc.subcore_barrier()

    rbufs = (zbuf, rbuf)
    rsems = (gsem0, gsem1)
    hop2 = [None] * ((N // NS) // ZB)
    for j in range((N // NS) // ZB):
        b = j % 2
        if j >= 2:
            hop2[j - 2].wait()
        stripe = pl.ds(sid * (N // NS) + j * ZB, ZB)
        pltpu.sync_copy(acc_sh.at[stripe], rbufs[b])
        hop2[j] = pltpu.async_copy(rbufs[b], out_hbm.at[cid, stripe],
                                   rsems[b])
    hop2[-2].wait()
    hop2[-1].wait()


def _sc_edge(asrc, adst, m, h_aug, src2, dst2):
    mesh = plsc.VectorSubcoreMesh(core_axis_name="c", subcore_axis_name="s",
                                  num_cores=NC, num_subcores=NS)
    kern = pl.kernel(
        _sc_body,
        out_type=jax.ShapeDtypeStruct((NC, N, D), jnp.float32),
        mesh=mesh,
        scratch_types=[
            pltpu.VMEM((N,), jnp.float32),
            pltpu.VMEM((N,), jnp.float32),
            pltpu.VMEM((L,), jnp.float32),
            pltpu.VMEM((NCHUNK, B), jnp.int32),
            pltpu.VMEM((NCHUNK, B), jnp.int32),
            pltpu.VMEM((EPT,), jnp.float32),
            pltpu.VMEM((B, D), jnp.float32),
            pltpu.VMEM((B, D), jnp.float32),
            pltpu.VMEM((B, D), jnp.float32),
            pltpu.VMEM((B, D), jnp.float32),
            pltpu.VMEM((ZB, D), jnp.float32),
            pltpu.VMEM((ZB, D), jnp.float32),
            pltpu.VMEM_SHARED((N, D), jnp.float32),
            pltpu.SemaphoreType.DMA,
            pltpu.SemaphoreType.DMA,
            pltpu.SemaphoreType.DMA,
            pltpu.SemaphoreType.DMA,
            pltpu.SemaphoreType.DMA,
            pltpu.SemaphoreType.DMA,
            pltpu.SemaphoreType.DMA,
            pltpu.SemaphoreType.DMA,
        ],
        compiler_params=pltpu.CompilerParams(needs_layout_passes=False,
                                             use_tc_tiling_on_sc=False),
    )
    return kern(asrc, adst, m, h_aug, src2, dst2)


def kernel(x, edge_index, W1, att_src1, att_dst1, b1, W2, att_src2, att_dst2, b2):
    f32 = jnp.float32
    src2 = edge_index[0].astype(jnp.int32).reshape(E // B, B)
    dst2 = edge_index[1].astype(jnp.int32).reshape(E // B, B)

    Wp1 = jnp.zeros((IN_CH_, D), f32).at[:, :HID_].set(W1)
    attc1 = (jnp.zeros((D, 2), f32)
             .at[:HID_, 0].set(att_src1)
             .at[:HID_, 1].set(att_dst1))
    b1p = jnp.zeros((1, D), f32).at[0, :HID_].set(b1)

    Wp2 = jnp.zeros((D, D), f32).at[:HID_, :OUT_].set(W2)
    attc2 = (jnp.zeros((D, 2), f32)
             .at[:OUT_, 0].set(att_src2)
             .at[:OUT_, 1].set(att_dst2))
    b2p = jnp.zeros((1, D), f32).at[0, :OUT_].set(b2)

    h1, a1, m1 = _dense1(x, Wp1, attc1)
    parts1 = _sc_edge(a1[:, 0], a1[:, 1], m1.reshape(L), h1, src2, dst2)
    h2, a2, m2 = _mid(parts1, h1, a1, m1, b1p, Wp2, attc2)
    parts2 = _sc_edge(a2[:, 0], a2[:, 1], m2.reshape(L), h2, src2, dst2)
    g2 = _fin2(parts2, h2, a2, m2, b2p)
    return (g2[:, :OUT_], edge_index)
